# baseline (speedup 1.0000x reference)
import jax
import jax.numpy as jnp
from jax.experimental import pallas as pl
from jax.experimental.pallas import tpu as pltpu

_NCHUNKS = 8


def _copy_kernel(in_hbm, out_hbm, sems):
    copies = []
    for i in range(_NCHUNKS):
        c = pltpu.make_async_copy(in_hbm.at[i], out_hbm.at[i], sems.at[i])
        c.start()
        copies.append(c)
    for c in copies:
        c.wait()


def kernel(attention_mask, inputs_embeds):
    del attention_mask
    b, s, h = inputs_embeds.shape
    x = inputs_embeds.reshape(_NCHUNKS, (b * s) // _NCHUNKS, h)
    out = pl.pallas_call(
        _copy_kernel,
        out_shape=jax.ShapeDtypeStruct(x.shape, x.dtype),
        in_specs=[pl.BlockSpec(memory_space=pl.ANY)],
        out_specs=pl.BlockSpec(memory_space=pl.ANY),
        scratch_shapes=[pltpu.SemaphoreType.DMA((_NCHUNKS,))],
    )(x)
    return out.reshape(b, s, h)


# speedup vs baseline: 49.1336x; 49.1336x over previous
import jax
import jax.numpy as jnp
from jax.experimental import pallas as pl

_BLOCK_ROWS = 1024


def _copy_kernel(in_ref, out_ref):
    out_ref[...] = in_ref[...]


def kernel(attention_mask, inputs_embeds):
    del attention_mask
    b, s, h = inputs_embeds.shape
    rows = b * s
    x = inputs_embeds.reshape(rows, h)
    out = pl.pallas_call(
        _copy_kernel,
        out_shape=jax.ShapeDtypeStruct((rows, h), x.dtype),
        grid=(rows // _BLOCK_ROWS,),
        in_specs=[pl.BlockSpec((_BLOCK_ROWS, h), lambda i: (i, 0))],
        out_specs=pl.BlockSpec((_BLOCK_ROWS, h), lambda i: (i, 0)),
    )(x)
    return out.reshape(b, s, h)
